# baseline (speedup 1.0000x reference)
import jax
import jax.numpy as jnp
from jax.experimental import pallas as pl
from jax.experimental.pallas import tpu as pltpu

T = 7
N = 512
RNN = 256
EMB = 128
OUT = 5
NTS = 32
K = 128
V = 64

BN = 128
NB = N // BN
NS = NB * T
TSPLIT = 4
VSPLIT = 2
KQ = K // TSPLIT
VQ = V // VSPLIT
NBUF = 3


def _fused(nodes_ref, ttc_hbm, ttcv_hbm, m_ref,
           win_ref, bin_ref, wt_ref, bt_ref, wtv_ref, btv_ref,
           wih_ref, whh_ref, bg_ref, wout_ref, bout_ref,
           h0_ref, c0_ref,
           out_ref, hs_ref, cs_ref,
           ttc_buf, ttcv_buf, ttc_sem, ttcv_sem):
    nb = pl.program_id(0)
    f = pl.program_id(1)
    s = nb * T + f
    n0 = nb * BN

    def copies(s2, buf2):
        f2 = s2 % T
        m0 = (s2 // T) * BN
        cps = []
        for q in range(TSPLIT):
            cps.append(pltpu.make_async_copy(
                ttc_hbm.at[pl.ds(f2, 1), pl.ds(m0, BN), pl.ds(q * KQ, KQ), :],
                ttc_buf.at[pl.ds(buf2, 1), :, pl.ds(q * KQ, KQ), :],
                ttc_sem.at[buf2, q]))
        for q in range(VSPLIT):
            cps.append(pltpu.make_async_copy(
                ttcv_hbm.at[pl.ds(f2, 1), pl.ds(m0, BN), pl.ds(q * VQ, VQ), :],
                ttcv_buf.at[pl.ds(buf2, 1), :, pl.ds(q * VQ, VQ), :],
                ttcv_sem.at[buf2, q]))
        return cps

    @pl.when(s == 0)
    def _():
        hs_ref[...] = h0_ref[...]
        cs_ref[...] = c0_ref[...]
        for j in range(NBUF - 1):
            for cp in copies(j, j):
                cp.start()

    @pl.when(s + NBUF - 1 < NS)
    def _():
        for cp in copies(s + NBUF - 1, (s + NBUF - 1) % NBUF):
            cp.start()

    buf = s % NBUF
    for cp in copies(s, buf):
        cp.wait()

    social = jnp.max(ttc_buf[buf], axis=1)
    social_veh = jnp.max(ttcv_buf[buf], axis=1)

    nodes = nodes_ref[f, pl.ds(n0, BN), :]
    inp_emb = jax.nn.relu(
        jnp.dot(nodes, win_ref[...], preferred_element_type=jnp.float32)
        + bin_ref[...])
    t_emb = jax.nn.relu(
        jnp.dot(social, wt_ref[...], preferred_element_type=jnp.float32)
        + bt_ref[...])
    tv_emb = jax.nn.relu(
        jnp.dot(social_veh, wtv_ref[...], preferred_element_type=jnp.float32)
        + btv_ref[...])
    concat = jnp.concatenate([inp_emb, t_emb, tv_emb], axis=1)

    h = hs_ref[pl.ds(n0, BN), :]
    c = cs_ref[pl.ds(n0, BN), :]
    gates = (jnp.dot(concat, wih_ref[...], preferred_element_type=jnp.float32)
             + jnp.dot(h, whh_ref[...], preferred_element_type=jnp.float32)
             + bg_ref[...])
    i_g = jax.nn.sigmoid(gates[:, 0:RNN])
    f_g = jax.nn.sigmoid(gates[:, RNN:2 * RNN])
    g_g = jnp.tanh(gates[:, 2 * RNN:3 * RNN])
    o_g = jax.nn.sigmoid(gates[:, 3 * RNN:4 * RNN])
    c_new = f_g * c + i_g * g_g
    h_new = o_g * jnp.tanh(c_new)

    out_raw = (jnp.dot(h_new, wout_ref[...], preferred_element_type=jnp.float32)
               + bout_ref[...])

    m = m_ref[f, pl.ds(n0, BN), :]
    out_ref[f, pl.ds(n0, BN), :] = m * out_raw
    hs_ref[pl.ds(n0, BN), :] = h + m * (h_new - h)
    cs_ref[pl.ds(n0, BN), :] = c + m * (c_new - c)


def kernel(input_data, grids, hidden_states, cell_states, mask, input_data_veh,
           grids_veh, mask_veh, grids_TTC, grids_TTC_veh,
           W_in, b_in, W_t, b_t, W_tv, b_tv, W_ih, W_hh, b_ih, b_hh,
           W_out, b_out):
    del grids, input_data_veh, grids_veh, mask_veh

    maskf = mask.astype(jnp.float32).reshape(T, N, 1)

    win = W_in.T
    wt = W_t.T
    wtv = W_tv.T
    wih = W_ih.T
    whh = W_hh.T
    bg = (b_ih + b_hh).reshape(1, 4 * RNN)
    wout = W_out.T
    bout = b_out.reshape(1, OUT)
    bin2 = b_in.reshape(1, EMB)
    bt2 = b_t.reshape(1, EMB)
    btv2 = b_tv.reshape(1, EMB)

    grid = (NB, T)

    def whole(shape):
        nd = len(shape)
        return pl.BlockSpec(shape, lambda nb, f, _nd=nd: (0,) * _nd)

    any_spec = pl.BlockSpec(memory_space=pltpu.MemorySpace.HBM)

    outputs, hs, cs = pl.pallas_call(
        _fused,
        grid=grid,
        in_specs=[
            whole((T, N, 2)),
            any_spec,
            any_spec,
            whole((T, N, 1)),
            whole((2, EMB)),
            whole((1, EMB)),
            whole((NTS, EMB)),
            whole((1, EMB)),
            whole((NTS, EMB)),
            whole((1, EMB)),
            whole((3 * EMB, 4 * RNN)),
            whole((RNN, 4 * RNN)),
            whole((1, 4 * RNN)),
            whole((RNN, OUT)),
            whole((1, OUT)),
            whole((N, RNN)),
            whole((N, RNN)),
        ],
        out_specs=[
            whole((T, N, OUT)),
            whole((N, RNN)),
            whole((N, RNN)),
        ],
        out_shape=[
            jax.ShapeDtypeStruct((T, N, OUT), jnp.float32),
            jax.ShapeDtypeStruct((N, RNN), jnp.float32),
            jax.ShapeDtypeStruct((N, RNN), jnp.float32),
        ],
        scratch_shapes=[
            pltpu.VMEM((NBUF, BN, K, NTS), jnp.float32),
            pltpu.VMEM((NBUF, BN, V, NTS), jnp.float32),
            pltpu.SemaphoreType.DMA((NBUF, TSPLIT)),
            pltpu.SemaphoreType.DMA((NBUF, VSPLIT)),
        ],
        compiler_params=pltpu.CompilerParams(
            dimension_semantics=("arbitrary", "arbitrary"),
        ),
    )(input_data, grids_TTC, grids_TTC_veh, maskf,
      win, bin2, wt, bt2, wtv, btv2, wih, whh, bg, wout, bout,
      hidden_states, cell_states)

    return outputs, hs, cs


# speedup vs baseline: 3.1642x; 3.1642x over previous
import jax
import jax.numpy as jnp
from jax.experimental import pallas as pl
from jax.experimental.pallas import tpu as pltpu

T = 7
N = 512
RNN = 256
EMB = 128
OUT = 5
NTS = 32
K = 128
V = 64

BN = 128
NB = N // BN


def _fused(nodes_ref, ttc_ref, ttcv_ref, m_ref,
           win_ref, bin_ref, wt_ref, bt_ref, wtv_ref, btv_ref,
           wih_ref, whh_ref, bg_ref, wout_ref, bout_ref,
           h0_ref, c0_ref,
           out_ref, hs_ref, cs_ref):
    nb = pl.program_id(0)
    f = pl.program_id(1)
    n0 = nb * BN

    @pl.when((nb == 0) & (f == 0))
    def _():
        hs_ref[...] = h0_ref[...]
        cs_ref[...] = c0_ref[...]

    social = jnp.max(ttc_ref[0], axis=2)
    social_veh = jnp.max(ttcv_ref[0], axis=2)

    nodes = nodes_ref[f, pl.ds(n0, BN), :]
    inp_emb = jax.nn.relu(
        jnp.dot(nodes, win_ref[...], preferred_element_type=jnp.float32)
        + bin_ref[...])
    t_emb = jax.nn.relu(
        jnp.dot(social, wt_ref[...], preferred_element_type=jnp.float32)
        + bt_ref[...])
    tv_emb = jax.nn.relu(
        jnp.dot(social_veh, wtv_ref[...], preferred_element_type=jnp.float32)
        + btv_ref[...])
    concat = jnp.concatenate([inp_emb, t_emb, tv_emb], axis=1)

    h = hs_ref[pl.ds(n0, BN), :]
    c = cs_ref[pl.ds(n0, BN), :]
    gates = (jnp.dot(concat, wih_ref[...], preferred_element_type=jnp.float32)
             + jnp.dot(h, whh_ref[...], preferred_element_type=jnp.float32)
             + bg_ref[...])
    i_g = jax.nn.sigmoid(gates[:, 0:RNN])
    f_g = jax.nn.sigmoid(gates[:, RNN:2 * RNN])
    g_g = jnp.tanh(gates[:, 2 * RNN:3 * RNN])
    o_g = jax.nn.sigmoid(gates[:, 3 * RNN:4 * RNN])
    c_new = f_g * c + i_g * g_g
    h_new = o_g * jnp.tanh(c_new)

    out_raw = (jnp.dot(h_new, wout_ref[...], preferred_element_type=jnp.float32)
               + bout_ref[...])

    m = m_ref[f, pl.ds(n0, BN), :]
    out_ref[f, pl.ds(n0, BN), :] = m * out_raw
    hs_ref[pl.ds(n0, BN), :] = h + m * (h_new - h)
    cs_ref[pl.ds(n0, BN), :] = c + m * (c_new - c)


def kernel(input_data, grids, hidden_states, cell_states, mask, input_data_veh,
           grids_veh, mask_veh, grids_TTC, grids_TTC_veh,
           W_in, b_in, W_t, b_t, W_tv, b_tv, W_ih, W_hh, b_ih, b_hh,
           W_out, b_out):
    del grids, input_data_veh, grids_veh, mask_veh

    ttc_t = jnp.transpose(grids_TTC, (0, 1, 3, 2))
    ttcv_t = jnp.transpose(grids_TTC_veh, (0, 1, 3, 2))
    maskf = mask.astype(jnp.float32).reshape(T, N, 1)

    win = W_in.T
    wt = W_t.T
    wtv = W_tv.T
    wih = W_ih.T
    whh = W_hh.T
    bg = (b_ih + b_hh).reshape(1, 4 * RNN)
    wout = W_out.T
    bout = b_out.reshape(1, OUT)
    bin2 = b_in.reshape(1, EMB)
    bt2 = b_t.reshape(1, EMB)
    btv2 = b_tv.reshape(1, EMB)

    grid = (NB, T)

    def nb_f4(nb, f):
        return (f, nb, 0, 0)

    def whole(shape):
        nd = len(shape)
        return pl.BlockSpec(shape, lambda nb, f, _nd=nd: (0,) * _nd)

    outputs, hs, cs = pl.pallas_call(
        _fused,
        grid=grid,
        in_specs=[
            whole((T, N, 2)),
            pl.BlockSpec((1, BN, NTS, K), nb_f4),
            pl.BlockSpec((1, BN, NTS, V), nb_f4),
            whole((T, N, 1)),
            whole((2, EMB)),
            whole((1, EMB)),
            whole((NTS, EMB)),
            whole((1, EMB)),
            whole((NTS, EMB)),
            whole((1, EMB)),
            whole((3 * EMB, 4 * RNN)),
            whole((RNN, 4 * RNN)),
            whole((1, 4 * RNN)),
            whole((RNN, OUT)),
            whole((1, OUT)),
            whole((N, RNN)),
            whole((N, RNN)),
        ],
        out_specs=[
            whole((T, N, OUT)),
            whole((N, RNN)),
            whole((N, RNN)),
        ],
        out_shape=[
            jax.ShapeDtypeStruct((T, N, OUT), jnp.float32),
            jax.ShapeDtypeStruct((N, RNN), jnp.float32),
            jax.ShapeDtypeStruct((N, RNN), jnp.float32),
        ],
        compiler_params=pltpu.CompilerParams(
            dimension_semantics=("arbitrary", "arbitrary"),
        ),
    )(input_data, ttc_t, ttcv_t, maskf,
      win, bin2, wt, bt2, wtv, btv2, wih, whh, bg, wout, bout,
      hidden_states, cell_states)

    return outputs, hs, cs


# speedup vs baseline: 4.9047x; 1.5501x over previous
import jax
import jax.numpy as jnp
from jax.experimental import pallas as pl
from jax.experimental.pallas import tpu as pltpu

T = 7
N = 512
RNN = 256
EMB = 128
OUT = 5
NTS = 32
K = 128
V = 64

BN = 128
NB = N // BN


def _fused(nodes_ref, ttc_ref, veh_ref, m_ref,
           win_ref, bin_ref, wt_ref, bt_ref, wtv_ref, btv_ref,
           wih_ref, whh_ref, bg_ref, wout_ref, bout_ref,
           h0_ref, c0_ref,
           out_ref, hs_ref, cs_ref,
           sv_ref):
    f = pl.program_id(0)
    nb = pl.program_id(1)
    n0 = nb * BN

    @pl.when((f == 0) & (nb == 0))
    def _():
        hs_ref[...] = h0_ref[...]
        cs_ref[...] = c0_ref[...]

    @pl.when(nb == 0)
    def _():
        sv_ref[...] = jnp.max(veh_ref[0], axis=0)

    social = jnp.max(ttc_ref[0], axis=2)
    svb = sv_ref[:, pl.ds(n0, BN)]

    nodes = nodes_ref[f, pl.ds(n0, BN), :]
    inp_emb = jax.nn.relu(
        jnp.dot(nodes, win_ref[...], preferred_element_type=jnp.float32)
        + bin_ref[...])
    t_emb = jax.nn.relu(
        jnp.dot(social, wt_ref[...], preferred_element_type=jnp.float32)
        + bt_ref[...])
    tv_emb = jax.nn.relu(
        jax.lax.dot_general(svb, wtv_ref[...], (((0,), (0,)), ((), ())),
                            preferred_element_type=jnp.float32)
        + btv_ref[...])
    concat = jnp.concatenate([inp_emb, t_emb, tv_emb], axis=1)

    h = hs_ref[pl.ds(n0, BN), :]
    c = cs_ref[pl.ds(n0, BN), :]
    gates = (jnp.dot(concat, wih_ref[...], preferred_element_type=jnp.float32)
             + jnp.dot(h, whh_ref[...], preferred_element_type=jnp.float32)
             + bg_ref[...])
    i_g = jax.nn.sigmoid(gates[:, 0:RNN])
    f_g = jax.nn.sigmoid(gates[:, RNN:2 * RNN])
    g_g = jnp.tanh(gates[:, 2 * RNN:3 * RNN])
    o_g = jax.nn.sigmoid(gates[:, 3 * RNN:4 * RNN])
    c_new = f_g * c + i_g * g_g
    h_new = o_g * jnp.tanh(c_new)

    out_raw = (jnp.dot(h_new, wout_ref[...], preferred_element_type=jnp.float32)
               + bout_ref[...])

    m = m_ref[f, pl.ds(n0, BN), :]
    out_ref[f, pl.ds(n0, BN), :] = m * out_raw
    hs_ref[pl.ds(n0, BN), :] = h + m * (h_new - h)
    cs_ref[pl.ds(n0, BN), :] = c + m * (c_new - c)


def kernel(input_data, grids, hidden_states, cell_states, mask, input_data_veh,
           grids_veh, mask_veh, grids_TTC, grids_TTC_veh,
           W_in, b_in, W_t, b_t, W_tv, b_tv, W_ih, W_hh, b_ih, b_hh,
           W_out, b_out):
    del grids, input_data_veh, grids_veh, mask_veh

    ttc_t = jnp.transpose(grids_TTC, (0, 1, 3, 2))
    veh_t = jnp.transpose(grids_TTC_veh, (0, 2, 3, 1))
    maskf = mask.astype(jnp.float32).reshape(T, N, 1)

    win = W_in.T
    wt = W_t.T
    wtv = W_tv.T
    wih = W_ih.T
    whh = W_hh.T
    bg = (b_ih + b_hh).reshape(1, 4 * RNN)
    wout = W_out.T
    bout = b_out.reshape(1, OUT)
    bin2 = b_in.reshape(1, EMB)
    bt2 = b_t.reshape(1, EMB)
    btv2 = b_tv.reshape(1, EMB)

    grid = (T, NB)

    def whole(shape):
        nd = len(shape)
        return pl.BlockSpec(shape, lambda f, nb, _nd=nd: (0,) * _nd)

    outputs, hs, cs = pl.pallas_call(
        _fused,
        grid=grid,
        in_specs=[
            whole((T, N, 2)),
            pl.BlockSpec((1, BN, NTS, K), lambda f, nb: (f, nb, 0, 0)),
            pl.BlockSpec((1, V, NTS, N), lambda f, nb: (f, 0, 0, 0)),
            whole((T, N, 1)),
            whole((2, EMB)),
            whole((1, EMB)),
            whole((NTS, EMB)),
            whole((1, EMB)),
            whole((NTS, EMB)),
            whole((1, EMB)),
            whole((3 * EMB, 4 * RNN)),
            whole((RNN, 4 * RNN)),
            whole((1, 4 * RNN)),
            whole((RNN, OUT)),
            whole((1, OUT)),
            whole((N, RNN)),
            whole((N, RNN)),
        ],
        out_specs=[
            whole((T, N, OUT)),
            whole((N, RNN)),
            whole((N, RNN)),
        ],
        out_shape=[
            jax.ShapeDtypeStruct((T, N, OUT), jnp.float32),
            jax.ShapeDtypeStruct((N, RNN), jnp.float32),
            jax.ShapeDtypeStruct((N, RNN), jnp.float32),
        ],
        scratch_shapes=[
            pltpu.VMEM((NTS, N), jnp.float32),
        ],
        compiler_params=pltpu.CompilerParams(
            dimension_semantics=("arbitrary", "arbitrary"),
        ),
    )(input_data, ttc_t, veh_t, maskf,
      win, bin2, wt, bt2, wtv, btv2, wih, whh, bg, wout, bout,
      hidden_states, cell_states)

    return outputs, hs, cs


# speedup vs baseline: 5.5889x; 1.1395x over previous
import jax
import jax.numpy as jnp
from jax.experimental import pallas as pl
from jax.experimental.pallas import tpu as pltpu

T = 7
N = 512
RNN = 256
EMB = 128
OUT = 5
NTS = 32
K = 128
V = 64

BN = 256
NB = N // BN


def _fused(nodes_ref, ttc_ref, veh_ref, m_ref,
           win_ref, bin_ref, wt_ref, bt_ref, wtv_ref, btv_ref,
           wih_ref, whh_ref, bg_ref, wout_ref, bout_ref,
           h0_ref, c0_ref,
           out_ref, hs_ref, cs_ref,
           sv_ref):
    f = pl.program_id(0)
    nb = pl.program_id(1)
    n0 = nb * BN

    @pl.when((f == 0) & (nb == 0))
    def _():
        hs_ref[...] = h0_ref[...]
        cs_ref[...] = c0_ref[...]

    @pl.when(nb == 0)
    def _():
        sv_ref[...] = jnp.max(veh_ref[0], axis=0)

    social = jnp.max(ttc_ref[0], axis=2)
    svb = sv_ref[:, pl.ds(n0, BN)]

    nodes = nodes_ref[f, pl.ds(n0, BN), :]
    inp_emb = jax.nn.relu(
        jnp.dot(nodes, win_ref[...], preferred_element_type=jnp.float32)
        + bin_ref[...])
    t_emb = jax.nn.relu(
        jnp.dot(social, wt_ref[...], preferred_element_type=jnp.float32)
        + bt_ref[...])
    tv_emb = jax.nn.relu(
        jax.lax.dot_general(svb, wtv_ref[...], (((0,), (0,)), ((), ())),
                            preferred_element_type=jnp.float32)
        + btv_ref[...])
    concat = jnp.concatenate([inp_emb, t_emb, tv_emb], axis=1)

    h = hs_ref[pl.ds(n0, BN), :]
    c = cs_ref[pl.ds(n0, BN), :]
    gates = (jnp.dot(concat, wih_ref[...], preferred_element_type=jnp.float32)
             + jnp.dot(h, whh_ref[...], preferred_element_type=jnp.float32)
             + bg_ref[...])
    i_g = jax.nn.sigmoid(gates[:, 0:RNN])
    f_g = jax.nn.sigmoid(gates[:, RNN:2 * RNN])
    g_g = jnp.tanh(gates[:, 2 * RNN:3 * RNN])
    o_g = jax.nn.sigmoid(gates[:, 3 * RNN:4 * RNN])
    c_new = f_g * c + i_g * g_g
    h_new = o_g * jnp.tanh(c_new)

    out_raw = (jnp.dot(h_new, wout_ref[...], preferred_element_type=jnp.float32)
               + bout_ref[...])

    m = m_ref[f, pl.ds(n0, BN), :]
    out_ref[f, pl.ds(n0, BN), :] = m * out_raw
    hs_ref[pl.ds(n0, BN), :] = h + m * (h_new - h)
    cs_ref[pl.ds(n0, BN), :] = c + m * (c_new - c)


def kernel(input_data, grids, hidden_states, cell_states, mask, input_data_veh,
           grids_veh, mask_veh, grids_TTC, grids_TTC_veh,
           W_in, b_in, W_t, b_t, W_tv, b_tv, W_ih, W_hh, b_ih, b_hh,
           W_out, b_out):
    del grids, input_data_veh, grids_veh, mask_veh

    ttc_t = jnp.transpose(grids_TTC, (0, 1, 3, 2))
    veh_t = jnp.transpose(grids_TTC_veh, (0, 2, 3, 1))
    maskf = mask.astype(jnp.float32).reshape(T, N, 1)

    win = W_in.T
    wt = W_t.T
    wtv = W_tv.T
    wih = W_ih.T
    whh = W_hh.T
    bg = (b_ih + b_hh).reshape(1, 4 * RNN)
    wout = W_out.T
    bout = b_out.reshape(1, OUT)
    bin2 = b_in.reshape(1, EMB)
    bt2 = b_t.reshape(1, EMB)
    btv2 = b_tv.reshape(1, EMB)

    grid = (T, NB)

    def whole(shape):
        nd = len(shape)
        return pl.BlockSpec(shape, lambda f, nb, _nd=nd: (0,) * _nd)

    outputs, hs, cs = pl.pallas_call(
        _fused,
        grid=grid,
        in_specs=[
            whole((T, N, 2)),
            pl.BlockSpec((1, BN, NTS, K), lambda f, nb: (f, nb, 0, 0)),
            pl.BlockSpec((1, V, NTS, N), lambda f, nb: (f, 0, 0, 0)),
            whole((T, N, 1)),
            whole((2, EMB)),
            whole((1, EMB)),
            whole((NTS, EMB)),
            whole((1, EMB)),
            whole((NTS, EMB)),
            whole((1, EMB)),
            whole((3 * EMB, 4 * RNN)),
            whole((RNN, 4 * RNN)),
            whole((1, 4 * RNN)),
            whole((RNN, OUT)),
            whole((1, OUT)),
            whole((N, RNN)),
            whole((N, RNN)),
        ],
        out_specs=[
            whole((T, N, OUT)),
            whole((N, RNN)),
            whole((N, RNN)),
        ],
        out_shape=[
            jax.ShapeDtypeStruct((T, N, OUT), jnp.float32),
            jax.ShapeDtypeStruct((N, RNN), jnp.float32),
            jax.ShapeDtypeStruct((N, RNN), jnp.float32),
        ],
        scratch_shapes=[
            pltpu.VMEM((NTS, N), jnp.float32),
        ],
        compiler_params=pltpu.CompilerParams(
            dimension_semantics=("arbitrary", "arbitrary"),
        ),
    )(input_data, ttc_t, veh_t, maskf,
      win, bin2, wt, bt2, wtv, btv2, wih, whh, bg, wout, bout,
      hidden_states, cell_states)

    return outputs, hs, cs


# speedup vs baseline: 6.3173x; 1.1303x over previous
import jax
import jax.numpy as jnp
from jax.experimental import pallas as pl
from jax.experimental.pallas import tpu as pltpu

T = 7
N = 512
RNN = 256
EMB = 128
OUT = 5
NTS = 32
K = 128
V = 64

BN = 512
NB = N // BN


def _fused(nodes_ref, ttc_ref, veh_ref, m_ref,
           win_ref, bin_ref, wt_ref, bt_ref, wtv_ref, btv_ref,
           wih_ref, whh_ref, bg_ref, wout_ref, bout_ref,
           h0_ref, c0_ref,
           out_ref, hs_ref, cs_ref,
           sv_ref):
    f = pl.program_id(0)
    nb = pl.program_id(1)
    n0 = nb * BN

    @pl.when((f == 0) & (nb == 0))
    def _():
        hs_ref[...] = h0_ref[...]
        cs_ref[...] = c0_ref[...]

    @pl.when(nb == 0)
    def _():
        sv_ref[...] = jnp.max(veh_ref[0], axis=0)

    social = jnp.max(ttc_ref[0], axis=2)
    svb = sv_ref[:, pl.ds(n0, BN)]

    nodes = nodes_ref[f, pl.ds(n0, BN), :]
    inp_emb = jax.nn.relu(
        jnp.dot(nodes, win_ref[...], preferred_element_type=jnp.float32)
        + bin_ref[...])
    t_emb = jax.nn.relu(
        jnp.dot(social, wt_ref[...], preferred_element_type=jnp.float32)
        + bt_ref[...])
    tv_emb = jax.nn.relu(
        jax.lax.dot_general(svb, wtv_ref[...], (((0,), (0,)), ((), ())),
                            preferred_element_type=jnp.float32)
        + btv_ref[...])
    concat = jnp.concatenate([inp_emb, t_emb, tv_emb], axis=1)

    h = hs_ref[pl.ds(n0, BN), :]
    c = cs_ref[pl.ds(n0, BN), :]
    gates = (jnp.dot(concat, wih_ref[...], preferred_element_type=jnp.float32)
             + jnp.dot(h, whh_ref[...], preferred_element_type=jnp.float32)
             + bg_ref[...])
    i_g = jax.nn.sigmoid(gates[:, 0:RNN])
    f_g = jax.nn.sigmoid(gates[:, RNN:2 * RNN])
    g_g = jnp.tanh(gates[:, 2 * RNN:3 * RNN])
    o_g = jax.nn.sigmoid(gates[:, 3 * RNN:4 * RNN])
    c_new = f_g * c + i_g * g_g
    h_new = o_g * jnp.tanh(c_new)

    out_raw = (jnp.dot(h_new, wout_ref[...], preferred_element_type=jnp.float32)
               + bout_ref[...])

    m = m_ref[f, pl.ds(n0, BN), :]
    out_ref[f, pl.ds(n0, BN), :] = m * out_raw
    hs_ref[pl.ds(n0, BN), :] = h + m * (h_new - h)
    cs_ref[pl.ds(n0, BN), :] = c + m * (c_new - c)


def kernel(input_data, grids, hidden_states, cell_states, mask, input_data_veh,
           grids_veh, mask_veh, grids_TTC, grids_TTC_veh,
           W_in, b_in, W_t, b_t, W_tv, b_tv, W_ih, W_hh, b_ih, b_hh,
           W_out, b_out):
    del grids, input_data_veh, grids_veh, mask_veh

    ttc_t = jnp.transpose(grids_TTC, (0, 1, 3, 2))
    veh_t = jnp.transpose(grids_TTC_veh, (0, 2, 3, 1))
    maskf = mask.astype(jnp.float32).reshape(T, N, 1)

    win = W_in.T
    wt = W_t.T
    wtv = W_tv.T
    wih = W_ih.T
    whh = W_hh.T
    bg = (b_ih + b_hh).reshape(1, 4 * RNN)
    wout = W_out.T
    bout = b_out.reshape(1, OUT)
    bin2 = b_in.reshape(1, EMB)
    bt2 = b_t.reshape(1, EMB)
    btv2 = b_tv.reshape(1, EMB)

    grid = (T, NB)

    def whole(shape):
        nd = len(shape)
        return pl.BlockSpec(shape, lambda f, nb, _nd=nd: (0,) * _nd)

    outputs, hs, cs = pl.pallas_call(
        _fused,
        grid=grid,
        in_specs=[
            whole((T, N, 2)),
            pl.BlockSpec((1, BN, NTS, K), lambda f, nb: (f, nb, 0, 0)),
            pl.BlockSpec((1, V, NTS, N), lambda f, nb: (f, 0, 0, 0)),
            whole((T, N, 1)),
            whole((2, EMB)),
            whole((1, EMB)),
            whole((NTS, EMB)),
            whole((1, EMB)),
            whole((NTS, EMB)),
            whole((1, EMB)),
            whole((3 * EMB, 4 * RNN)),
            whole((RNN, 4 * RNN)),
            whole((1, 4 * RNN)),
            whole((RNN, OUT)),
            whole((1, OUT)),
            whole((N, RNN)),
            whole((N, RNN)),
        ],
        out_specs=[
            whole((T, N, OUT)),
            whole((N, RNN)),
            whole((N, RNN)),
        ],
        out_shape=[
            jax.ShapeDtypeStruct((T, N, OUT), jnp.float32),
            jax.ShapeDtypeStruct((N, RNN), jnp.float32),
            jax.ShapeDtypeStruct((N, RNN), jnp.float32),
        ],
        scratch_shapes=[
            pltpu.VMEM((NTS, N), jnp.float32),
        ],
        compiler_params=pltpu.CompilerParams(
            dimension_semantics=("arbitrary", "arbitrary"),
        ),
    )(input_data, ttc_t, veh_t, maskf,
      win, bin2, wt, bt2, wtv, btv2, wih, whh, bg, wout, bout,
      hidden_states, cell_states)

    return outputs, hs, cs
